# baseline (speedup 1.0000x reference)
import functools

import jax
import jax.numpy as jnp
from jax import lax
from jax.experimental import pallas as pl
from jax.experimental.pallas import tpu as pltpu
from jax.experimental.pallas import tpu_sc as plsc

NUM_ROWS = 1000000
DIM = 32
B_TOKENS = 16384
SEQ = 26
B_FLAT = B_TOKENS * SEQ
OUT_ROWS = B_TOKENS * 32 * 4
CB = 2048
N_IN_BLOCKS = -(-NUM_ROWS // CB)
STEPS = -(-N_IN_BLOCKS // 4)
IMG_ROWS = STEPS * CB
LIN_ROWS = IMG_ROWS * 4

_info = plsc.get_sparse_core_info()
NC = _info.num_cores
NS = _info.num_subcores
NW = NC * NS
B_PER_W = B_FLAT // NW
CHUNK = 1024
N_CHUNKS = B_PER_W // CHUNK
NBUF = 2
DIV26_MUL = 80660
DIV26_SHIFT = 21


def _tr_body(a0_ref, a1_ref, a2_ref, a3_ref, out_ref):
    eye = jnp.eye(DIM, dtype=jnp.float32)
    dn = (((0,), (0,)), ((), ()))
    parts = [
        lax.dot_general(r[...], eye, dn, precision=lax.Precision.HIGHEST,
                        preferred_element_type=jnp.float32)
        for r in (a0_ref, a1_ref, a2_ref, a3_ref)
    ]
    out_ref[...] = jnp.concatenate(parts, axis=1)


def _chunk_spec(a):
    return pl.BlockSpec(
        (DIM, CB),
        lambda g, a=a: (0, jnp.minimum(g * 4 + a, N_IN_BLOCKS - 1)))


_transpose = pl.pallas_call(
    _tr_body,
    grid=(STEPS,),
    in_specs=[_chunk_spec(a) for a in range(4)],
    out_specs=pl.BlockSpec((CB, DIM * 4), lambda g: (g, 0)),
    out_shape=jax.ShapeDtypeStruct((IMG_ROWS, DIM * 4), jnp.float32),
)

_mesh = plsc.VectorSubcoreMesh(core_axis_name="c", subcore_axis_name="s")


@functools.partial(
    pl.kernel,
    mesh=_mesh,
    out_type=jax.ShapeDtypeStruct((OUT_ROWS, DIM), jnp.float32),
    compiler_params=pltpu.CompilerParams(use_tc_tiling_on_sc=False),
    scratch_types=[
        pltpu.VMEM((B_PER_W,), jnp.int32),
        pltpu.VMEM((N_CHUNKS, CHUNK), jnp.int32),
        [pltpu.VMEM((CHUNK, DIM), jnp.float32) for _ in range(NBUF)],
        pltpu.SemaphoreType.DMA,
        pltpu.SemaphoreType.DMA,
    ],
)
def _gather(idx_hbm, table_hbm, out_hbm, idx_v, pos_v, rows, gsem, osem):
    wid = lax.axis_index("s") * NC + lax.axis_index("c")
    base = wid * B_PER_W
    tbase = wid * (B_PER_W // SEQ)

    pltpu.sync_copy(idx_hbm.at[pl.ds(base, B_PER_W)], idx_v)

    iota16 = lax.iota(jnp.int32, 16)

    def pbody(mm, carry):
        c = mm // (CHUNK // 16)
        m = mm % (CHUNK // 16)
        off = c * CHUNK + m * 16
        i = idx_v[pl.ds(off, 16)]
        b = lax.shift_right_logical(i, 11)
        j = lax.bitwise_and(i, CB - 1)
        g = lax.shift_right_logical(b, 2)
        idx_v[pl.ds(off, 16)] = (g * CB + j) * 4 + lax.bitwise_and(b, 3)
        jl = iota16 + off
        t = lax.shift_right_logical(jl * DIV26_MUL, DIV26_SHIFT)
        s = jl - t * SEQ
        pos_v[c, pl.ds(m * 16, 16)] = ((tbase + t) * 32 + s) * 4
        return carry

    lax.fori_loop(0, N_CHUNKS * (CHUNK // 16), pbody, 0)

    def start_gather(c):
        return pltpu.async_copy(
            table_hbm.at[idx_v.at[pl.ds(c * CHUNK, CHUNK)]],
            rows[c % NBUF], gsem)

    def start_store(c):
        return pltpu.async_copy(
            rows[c % NBUF], out_hbm.at[pos_v.at[c]], osem)

    gathers = [start_gather(0)]
    stores = []
    for c in range(N_CHUNKS):
        if c + 1 < N_CHUNKS:
            if c >= 1:
                stores[c - 1].wait()
            gathers.append(start_gather(c + 1))
        gathers[c].wait()
        stores.append(start_store(c))
    stores[N_CHUNKS - 2].wait()
    stores[N_CHUNKS - 1].wait()


def kernel(x, mat):
    xf = x.reshape(B_FLAT)
    matT = mat.T
    mat4 = _transpose(matT, matT, matT, matT)
    mat_lin = mat4.reshape(LIN_ROWS, DIM)
    out_pad = _gather(xf, mat_lin)
    return out_pad.reshape(B_TOKENS, 32, 128)[:, :SEQ, :DIM]


# speedup vs baseline: 1.6724x; 1.6724x over previous
import functools

import jax
import jax.numpy as jnp
from jax import lax
from jax.experimental import pallas as pl
from jax.experimental.pallas import tpu as pltpu
from jax.experimental.pallas import tpu_sc as plsc

NUM_ROWS = 1000000
DIM = 32
B_TOKENS = 16384
SEQ = 26
B_FLAT = B_TOKENS * SEQ
OUT_ROWS = B_TOKENS * 32 * 4
CB = 2048
N_IN_BLOCKS = -(-NUM_ROWS // CB)
STEPS = -(-N_IN_BLOCKS // 4)
IMG_ROWS = STEPS * CB
LIN_ROWS = IMG_ROWS * 4

_info = plsc.get_sparse_core_info()
NC = _info.num_cores
NS = _info.num_subcores
NW = NC * NS
B_PER_W = B_FLAT // NW
CHUNK = 1024
N_CHUNKS = B_PER_W // CHUNK
NBUF = 2
DIV26_MUL = 80660
DIV26_SHIFT = 21


def _tr_body(a0_ref, a1_ref, a2_ref, a3_ref, out_ref):
    eye = jnp.eye(DIM, dtype=jnp.float32)
    dn = (((0,), (0,)), ((), ()))
    parts = [
        lax.dot_general(r[...], eye, dn, preferred_element_type=jnp.float32)
        for r in (a0_ref, a1_ref, a2_ref, a3_ref)
    ]
    out_ref[...] = jnp.concatenate(parts, axis=1)


def _chunk_spec(a):
    return pl.BlockSpec(
        (DIM, CB),
        lambda g, a=a: (0, jnp.minimum(g * 4 + a, N_IN_BLOCKS - 1)))


_transpose = pl.pallas_call(
    _tr_body,
    grid=(STEPS,),
    in_specs=[_chunk_spec(a) for a in range(4)],
    out_specs=pl.BlockSpec((CB, DIM * 4), lambda g: (g, 0)),
    out_shape=jax.ShapeDtypeStruct((IMG_ROWS, DIM * 4), jnp.float32),
)

_mesh = plsc.VectorSubcoreMesh(core_axis_name="c", subcore_axis_name="s")


@functools.partial(
    pl.kernel,
    mesh=_mesh,
    out_type=jax.ShapeDtypeStruct((OUT_ROWS, DIM), jnp.float32),
    compiler_params=pltpu.CompilerParams(use_tc_tiling_on_sc=False),
    scratch_types=[
        pltpu.VMEM((B_PER_W,), jnp.int32),
        pltpu.VMEM((N_CHUNKS, CHUNK), jnp.int32),
        [pltpu.VMEM((CHUNK, DIM), jnp.float32) for _ in range(NBUF)],
        pltpu.SemaphoreType.DMA,
        pltpu.SemaphoreType.DMA,
    ],
)
def _gather(idx_hbm, table_hbm, out_hbm, idx_v, pos_v, rows, gsem, osem):
    wid = lax.axis_index("s") * NC + lax.axis_index("c")
    base = wid * B_PER_W
    tbase = wid * (B_PER_W // SEQ)

    pltpu.sync_copy(idx_hbm.at[pl.ds(base, B_PER_W)], idx_v)

    iota16 = lax.iota(jnp.int32, 16)

    def pbody(mm, carry):
        c = mm // (CHUNK // 16)
        m = mm % (CHUNK // 16)
        off = c * CHUNK + m * 16
        i = idx_v[pl.ds(off, 16)]
        b = lax.shift_right_logical(i, 11)
        j = lax.bitwise_and(i, CB - 1)
        g = lax.shift_right_logical(b, 2)
        idx_v[pl.ds(off, 16)] = (g * CB + j) * 4 + lax.bitwise_and(b, 3)
        jl = iota16 + off
        t = lax.shift_right_logical(jl * DIV26_MUL, DIV26_SHIFT)
        s = jl - t * SEQ
        pos_v[c, pl.ds(m * 16, 16)] = ((tbase + t) * 32 + s) * 4
        return carry

    lax.fori_loop(0, N_CHUNKS * (CHUNK // 16), pbody, 0)

    def start_gather(c):
        return pltpu.async_copy(
            table_hbm.at[idx_v.at[pl.ds(c * CHUNK, CHUNK)]],
            rows[c % NBUF], gsem)

    def start_store(c):
        return pltpu.async_copy(
            rows[c % NBUF], out_hbm.at[pos_v.at[c]], osem)

    gathers = [start_gather(0)]
    stores = []
    for c in range(N_CHUNKS):
        if c + 1 < N_CHUNKS:
            if c >= 1:
                stores[c - 1].wait()
            gathers.append(start_gather(c + 1))
        gathers[c].wait()
        stores.append(start_store(c))
    stores[N_CHUNKS - 2].wait()
    stores[N_CHUNKS - 1].wait()


def kernel(x, mat):
    xf = x.reshape(B_FLAT)
    matT = mat.T
    mat4 = _transpose(matT, matT, matT, matT)
    mat_lin = mat4.reshape(LIN_ROWS, DIM)
    out_pad = _gather(xf, mat_lin)
    return out_pad.reshape(B_TOKENS, 32, 128)[:, :SEQ, :DIM]
